# baseline (speedup 1.0000x reference)
import jax
import jax.numpy as jnp
from jax import lax
from jax.experimental import pallas as pl
from jax.experimental.pallas import tpu as pltpu
from jax.experimental.pallas import tpu_sc as plsc

NUM_CLASSES = 16
TIMESTEPS = 300
N_TOKENS = 131072

NUM_CORES = 2
NUM_SUBCORES = 16
LANES = 16
NUM_WORKERS = NUM_CORES * NUM_SUBCORES
TOK_PER_WORKER = N_TOKENS // NUM_WORKERS
CHUNK = 2048
TAB = 304


def _sc_body(x0_hbm, t_hbm, qt_hbm, out_hbm,
             xa_v, xb_v, t_v, q_v, dotab_v,
             sem_ia, sem_ib, sem_oa, sem_ob, sem_t):
    cid = lax.axis_index("c")
    sid = lax.axis_index("s")
    wid = sid * NUM_CORES + cid
    base = wid * TOK_PER_WORKER

    in_a = pltpu.async_copy(x0_hbm.at[:, pl.ds(base, CHUNK)], xa_v, sem_ia)
    in_b = pltpu.async_copy(x0_hbm.at[:, pl.ds(base + CHUNK, CHUNK)], xb_v, sem_ib)
    t_cp = pltpu.async_copy(t_hbm.at[pl.ds(base, TOK_PER_WORKER)], t_v, sem_t)

    pltpu.sync_copy(qt_hbm.at[0], q_v)
    for g in range(TIMESTEPS // LANES):
        dotab_v[pl.ds(g * LANES, LANES)] = q_v[0, pl.ds(g * LANES, LANES)]
        dotab_v[pl.ds(TAB + g * LANES, LANES)] = q_v[1, pl.ds(g * LANES, LANES)]
    tail = TIMESTEPS - LANES
    dotab_v[pl.ds(tail, LANES)] = q_v[0, pl.ds(tail, LANES)]
    dotab_v[pl.ds(TAB + tail, LANES)] = q_v[1, pl.ds(tail, LANES)]

    t_cp.wait()

    def _compute(c, x_v):
        def block(g, carry):
            tvec = t_v[pl.ds(c * CHUNK + g * LANES, LANES)]
            d = plsc.load_gather(dotab_v, [tvec])
            o = plsc.load_gather(dotab_v, [tvec + TAB])
            w = d - o
            rs = [x_v[j, pl.ds(g * LANES, LANES)] for j in range(NUM_CLASSES)]
            s = rs[0]
            for j in range(1, NUM_CLASSES):
                s = s + rs[j]
            os = o * s
            for j in range(NUM_CLASSES):
                x_v[j, pl.ds(g * LANES, LANES)] = w * rs[j] + os
            return carry

        lax.fori_loop(0, CHUNK // LANES, block, 0)

    in_a.wait()
    _compute(0, xa_v)
    out_a = pltpu.async_copy(xa_v, out_hbm.at[:, pl.ds(base, CHUNK)], sem_oa)
    in_b.wait()
    _compute(1, xb_v)
    out_b = pltpu.async_copy(xb_v, out_hbm.at[:, pl.ds(base + CHUNK, CHUNK)], sem_ob)
    out_a.wait()
    out_b.wait()


@jax.jit
def _run(x0t, t, qtp):
    mesh = plsc.VectorSubcoreMesh(core_axis_name="c", subcore_axis_name="s")
    return pl.kernel(
        _sc_body,
        out_type=jax.ShapeDtypeStruct((NUM_CLASSES, N_TOKENS), jnp.float32),
        mesh=mesh,
        scratch_types=[
            pltpu.VMEM((NUM_CLASSES, CHUNK), jnp.float32),
            pltpu.VMEM((NUM_CLASSES, CHUNK), jnp.float32),
            pltpu.VMEM((TOK_PER_WORKER,), jnp.int32),
            pltpu.VMEM((NUM_CLASSES, TIMESTEPS), jnp.float32),
            pltpu.VMEM((2 * TAB,), jnp.float32),
            pltpu.SemaphoreType.DMA,
            pltpu.SemaphoreType.DMA,
            pltpu.SemaphoreType.DMA,
            pltpu.SemaphoreType.DMA,
            pltpu.SemaphoreType.DMA,
        ],
        compiler_params=pltpu.CompilerParams(needs_layout_passes=False),
    )(x0t, t, qtp)


def kernel(x0, t, Qt_bar):
    out_t = _run(x0.T, t.astype(jnp.int32), jnp.transpose(Qt_bar, (1, 2, 0)))
    return out_t.T


# speedup vs baseline: 1.0702x; 1.0702x over previous
import jax
import jax.numpy as jnp
from jax import lax
from jax.experimental import pallas as pl
from jax.experimental.pallas import tpu as pltpu
from jax.experimental.pallas import tpu_sc as plsc

NUM_CLASSES = 16
TIMESTEPS = 300
N_TOKENS = 131072

NUM_CORES = 2
NUM_SUBCORES = 16
LANES = 16
NUM_WORKERS = NUM_CORES * NUM_SUBCORES
TOK_PER_WORKER = N_TOKENS // NUM_WORKERS
CHUNK = 2048
TAB = 304


def _sc_body(x0_hbm, t_hbm, qt_hbm, out_hbm,
             xa_v, xb_v, t_v, q_v, dotab_v,
             sem_ia, sem_ib, sem_oa, sem_ob, sem_t):
    cid = lax.axis_index("c")
    sid = lax.axis_index("s")
    wid = sid * NUM_CORES + cid
    base = wid * TOK_PER_WORKER

    in_a = pltpu.async_copy(x0_hbm.at[:, pl.ds(base, CHUNK)], xa_v, sem_ia)
    in_b = pltpu.async_copy(x0_hbm.at[:, pl.ds(base + CHUNK, CHUNK)], xb_v, sem_ib)
    t_cp = pltpu.async_copy(t_hbm.at[pl.ds(base, TOK_PER_WORKER)], t_v, sem_t)

    pltpu.sync_copy(qt_hbm.at[0], q_v)
    for g in range(TIMESTEPS // LANES):
        dotab_v[pl.ds(g * LANES, LANES)] = q_v[0, pl.ds(g * LANES, LANES)]
        dotab_v[pl.ds(TAB + g * LANES, LANES)] = q_v[1, pl.ds(g * LANES, LANES)]
    tail = TIMESTEPS - LANES
    dotab_v[pl.ds(tail, LANES)] = q_v[0, pl.ds(tail, LANES)]
    dotab_v[pl.ds(TAB + tail, LANES)] = q_v[1, pl.ds(tail, LANES)]

    t_cp.wait()

    def _compute(c, x_v):
        @plsc.parallel_loop(0, CHUNK // LANES, unroll=4)
        def block(g):
            tvec = t_v[pl.ds(c * CHUNK + g * LANES, LANES)]
            d = plsc.load_gather(dotab_v, [tvec])
            o = plsc.load_gather(dotab_v, [tvec + TAB])
            w = d - o
            rs = [x_v[j, pl.ds(g * LANES, LANES)] for j in range(NUM_CLASSES)]
            acc = rs
            while len(acc) > 1:
                acc = [a + b for a, b in zip(acc[::2], acc[1::2])]
            os = o * acc[0]
            for j in range(NUM_CLASSES):
                x_v[j, pl.ds(g * LANES, LANES)] = w * rs[j] + os

    in_a.wait()
    _compute(0, xa_v)
    out_a = pltpu.async_copy(xa_v, out_hbm.at[:, pl.ds(base, CHUNK)], sem_oa)
    in_b.wait()
    _compute(1, xb_v)
    out_b = pltpu.async_copy(xb_v, out_hbm.at[:, pl.ds(base + CHUNK, CHUNK)], sem_ob)
    out_a.wait()
    out_b.wait()


@jax.jit
def _run(x0t, t, qtp):
    mesh = plsc.VectorSubcoreMesh(core_axis_name="c", subcore_axis_name="s")
    return pl.kernel(
        _sc_body,
        out_type=jax.ShapeDtypeStruct((NUM_CLASSES, N_TOKENS), jnp.float32),
        mesh=mesh,
        scratch_types=[
            pltpu.VMEM((NUM_CLASSES, CHUNK), jnp.float32),
            pltpu.VMEM((NUM_CLASSES, CHUNK), jnp.float32),
            pltpu.VMEM((TOK_PER_WORKER,), jnp.int32),
            pltpu.VMEM((NUM_CLASSES, TIMESTEPS), jnp.float32),
            pltpu.VMEM((2 * TAB,), jnp.float32),
            pltpu.SemaphoreType.DMA,
            pltpu.SemaphoreType.DMA,
            pltpu.SemaphoreType.DMA,
            pltpu.SemaphoreType.DMA,
            pltpu.SemaphoreType.DMA,
        ],
        compiler_params=pltpu.CompilerParams(needs_layout_passes=False),
    )(x0t, t, qtp)


def kernel(x0, t, Qt_bar):
    out_t = _run(x0.T, t.astype(jnp.int32), jnp.transpose(Qt_bar, (1, 2, 0)))
    return out_t.T
